# baseline (speedup 1.0000x reference)
import functools
import math

import jax
import jax.numpy as jnp
from jax import lax
from jax.experimental import pallas as pl
from jax.experimental.pallas import tpu as pltpu
from jax.experimental.pallas import tpu_sc as plsc

B = 1024
D = 64
N_ROWS = 1000000
KP1 = 512
T = 0.07

NC = 2
NS = 16
NW = NC * NS
BPW = B // NW
NBUF = 4
LPU = 256
NUNIT = BPW * KP1 // LPU
CHUNK = 128
NCHUNK = LPU // CHUNK


def _sc_body(x_hbm, y_hbm, mem_hbm, idx_hbm, out_hbm, sums_hbm,
             x_v, y_v, idx_v, rows0, rows1, rows2, rows3, out_v, sum_v,
             sem0, sem1, sem2, sem3):
    wid = lax.axis_index("s") * NC + lax.axis_index("c")
    b0 = wid * BPW

    pltpu.sync_copy(x_hbm.at[pl.ds(b0, BPW)], x_v)
    pltpu.sync_copy(y_hbm.at[pl.ds(b0, BPW)], y_v)
    pltpu.sync_copy(idx_hbm.at[pl.ds(b0, BPW)], idx_v)

    lanes = lax.iota(jnp.int32, 16)

    zeros16 = jnp.zeros((16,), jnp.int32)
    for h in range(BPW // 16):
        yv = y_v[pl.ds(h * 16, 16)]
        plsc.store_scatter(idx_v, [h * 16 + lanes, zeros16], yv)

    rows_bufs = (rows0, rows1, rows2, rows3)
    sems = (sem0, sem1, sem2, sem3)

    def _fire(u, buf, sem):
        lb = u >> 1
        kb = (u & 1) * LPU
        for c in range(NCHUNK):
            pltpu.async_copy(
                mem_hbm.at[idx_v.at[lb, pl.ds(kb + c * CHUNK, CHUNK)]],
                buf.at[pl.ds(c * CHUNK, CHUNK)],
                sem,
            )

    def _drain(u, buf, sem):
        lb = u >> 1
        kb = (u & 1) * LPU
        for c in range(NCHUNK):
            pltpu.make_async_copy(
                mem_hbm.at[idx_v.at[lb, pl.ds(kb + c * CHUNK, CHUNK)]],
                buf.at[pl.ds(c * CHUNK, CHUNK)],
                sem,
            ).wait()

    inv_t = jnp.float32(1.0 / T)

    def _compute(u, buf, acc):
        lb = u >> 1
        kb = (u & 1) * LPU
        xr = [x_v[lb, pl.ds(q * 16, 16)] for q in range(D // 16)]

        def _group(g, acc_in):
            ridx = g * 16 + lanes
            accs = [jnp.zeros((16,), jnp.float32) for _ in range(4)]
            for d0 in range(0, D, 2):
                rv0 = plsc.load_gather(buf, [ridx, jnp.full((16,), d0, jnp.int32)])
                rv1 = plsc.load_gather(buf, [ridx, jnp.full((16,), d0 + 1, jnp.int32)])
                r0, r1 = plsc.unpack(
                    plsc.pack(rv0, rv1, format=plsc.PackFormat.INTERLEAVED),
                    format=plsc.PackFormat.INTERLEAVED)
                j = d0 & 2
                accs[j] = accs[j] + r0 * xr[d0 // 16][d0 % 16]
                accs[j + 1] = accs[j + 1] + r1 * xr[(d0 + 1) // 16][(d0 + 1) % 16]
            dot = (accs[0] + accs[1]) + (accs[2] + accs[3])
            ev = jnp.exp(dot * inv_t)
            out_v[lb, pl.ds(kb + g * 16, 16)] = ev
            return acc_in + ev
        return lax.fori_loop(0, LPU // 16, _group, acc)

    for p in range(NBUF - 1):
        _fire(p, rows_bufs[p], sems[p])

    def _step(i, acc):
        for j in range(NBUF):
            u = NBUF * i + j
            nxt = u + NBUF - 1
            jn = (j + NBUF - 1) % NBUF
            if j == 0:
                _fire(nxt, rows_bufs[jn], sems[jn])
            else:
                @pl.when(i < NUNIT // NBUF - 1)
                def _():
                    _fire(nxt, rows_bufs[jn], sems[jn])
            _drain(u, rows_bufs[j], sems[j])
            acc = _compute(u, rows_bufs[j], acc)
        return acc

    acc = lax.fori_loop(0, NUNIT // NBUF, _step, jnp.zeros((16,), jnp.float32))

    sum_v[...] = acc
    pltpu.sync_copy(out_v, out_hbm.at[pl.ds(b0, BPW)])
    pltpu.sync_copy(sum_v, sums_hbm.at[wid])


def _normalize_body(e_ref, s_ref, o_ref):
    total = jnp.sum(s_ref[...])
    scale = jnp.float32(B * KP1 / N_ROWS) / total
    o_ref[...] = e_ref[...] * scale


@jax.jit
def kernel(x, y, memory, idx):
    mesh = plsc.VectorSubcoreMesh(core_axis_name="c", subcore_axis_name="s")
    sc = pl.kernel(
        _sc_body,
        out_type=(
            jax.ShapeDtypeStruct((B, KP1), jnp.float32),
            jax.ShapeDtypeStruct((NW, 16), jnp.float32),
        ),
        mesh=mesh,
        compiler_params=pltpu.CompilerParams(
            use_tc_tiling_on_sc=False, needs_layout_passes=False),
        scratch_types=[
            pltpu.VMEM((BPW, D), jnp.float32),
            pltpu.VMEM((BPW,), jnp.int32),
            pltpu.VMEM((BPW, KP1), jnp.int32),
            pltpu.VMEM((LPU, D), jnp.float32),
            pltpu.VMEM((LPU, D), jnp.float32),
            pltpu.VMEM((LPU, D), jnp.float32),
            pltpu.VMEM((LPU, D), jnp.float32),
            pltpu.VMEM((BPW, KP1), jnp.float32),
            pltpu.VMEM((16,), jnp.float32),
            pltpu.SemaphoreType.DMA,
            pltpu.SemaphoreType.DMA,
            pltpu.SemaphoreType.DMA,
            pltpu.SemaphoreType.DMA,
        ],
    )
    from jax.experimental import layout as jlayout
    memory = jlayout.with_layout_constraint(
        memory, jlayout.Layout((0, 1), tiling=((8,), (1024,))))
    x_r = x.astype(jnp.bfloat16).astype(jnp.float32)
    e, sums = sc(x_r, y.astype(jnp.int32), memory, idx.astype(jnp.int32))
    out = pl.pallas_call(
        _normalize_body,
        out_shape=jax.ShapeDtypeStruct((B, KP1), jnp.float32),
    )(e, sums)
    return out


# speedup vs baseline: 1.9457x; 1.9457x over previous
import functools
import math

import jax
import jax.numpy as jnp
from jax import lax
from jax.experimental import pallas as pl
from jax.experimental.pallas import tpu as pltpu
from jax.experimental.pallas import tpu_sc as plsc

B = 1024
D = 64
N_ROWS = 1000000
KP1 = 512
T = 0.07

NC = 2
NS = 16
NW = NC * NS
BPW = B // NW
NBUF = 4
LPU = 256
NUNIT = BPW * KP1 // LPU
CHUNK = 128
NCHUNK = LPU // CHUNK


def _sc_body(x_hbm, y_hbm, mem_hbm, idx_hbm, out_hbm, sums_hbm,
             x_v, y_v, idx_v, rows0, rows1, rows2, rows3, out_v, sum_v,
             sem0, sem1, sem2, sem3):
    wid = lax.axis_index("s") * NC + lax.axis_index("c")
    b0 = wid * BPW

    pltpu.sync_copy(x_hbm.at[pl.ds(b0, BPW)], x_v)
    pltpu.sync_copy(y_hbm.at[pl.ds(b0, BPW)], y_v)
    pltpu.sync_copy(idx_hbm.at[pl.ds(b0, BPW)], idx_v)

    lanes = lax.iota(jnp.int32, 16)

    zeros16 = jnp.zeros((16,), jnp.int32)
    for h in range(BPW // 16):
        yv = y_v[pl.ds(h * 16, 16)]
        plsc.store_scatter(idx_v, [h * 16 + lanes, zeros16], yv)

    rows_bufs = (rows0, rows1, rows2, rows3)
    sems = (sem0, sem1, sem2, sem3)

    def _fire(u, buf, sem):
        lb = u >> 1
        kb = (u & 1) * LPU
        for c in range(NCHUNK):
            pltpu.async_copy(
                mem_hbm.at[idx_v.at[lb, pl.ds(kb + c * CHUNK, CHUNK)]],
                buf.at[pl.ds(c * CHUNK, CHUNK)],
                sem,
            )

    def _drain(u, buf, sem):
        lb = u >> 1
        kb = (u & 1) * LPU
        for c in range(NCHUNK):
            pltpu.make_async_copy(
                mem_hbm.at[idx_v.at[lb, pl.ds(kb + c * CHUNK, CHUNK)]],
                buf.at[pl.ds(c * CHUNK, CHUNK)],
                sem,
            ).wait()

    inv_t = jnp.float32(1.0 / T)

    def _compute(u, buf, acc):
        lb = u >> 1
        kb = (u & 1) * LPU
        xr = [x_v[lb, pl.ds(q * 16, 16)] for q in range(D // 16)]

        def _group(g, acc_in):
            dots = []
            for j in range(16):
                row = g * 16 + j
                r = [buf[row, pl.ds(q * 16, 16)] for q in range(D // 16)]
                u0, u1 = plsc.unpack(
                    plsc.pack(r[0], r[1], format=plsc.PackFormat.INTERLEAVED),
                    format=plsc.PackFormat.INTERLEAVED)
                u2, u3 = plsc.unpack(
                    plsc.pack(r[2], r[3], format=plsc.PackFormat.INTERLEAVED),
                    format=plsc.PackFormat.INTERLEAVED)
                m = (u0 * xr[0] + u1 * xr[1]) + (u2 * xr[2] + u3 * xr[3])
                dots.append(jnp.sum(m))
            dot = jnp.zeros((16,), jnp.float32)
            for j in range(16):
                dot = jnp.where(lanes == j, dots[j], dot)
            ev = jnp.exp(dot * inv_t)
            out_v[lb, pl.ds(kb + g * 16, 16)] = ev
            return acc_in + ev
        return lax.fori_loop(0, LPU // 16, _group, acc)

    for p in range(NBUF - 1):
        _fire(p, rows_bufs[p], sems[p])

    def _step(i, acc):
        for j in range(NBUF):
            u = NBUF * i + j
            nxt = u + NBUF - 1
            jn = (j + NBUF - 1) % NBUF
            if j == 0:
                _fire(nxt, rows_bufs[jn], sems[jn])
            else:
                @pl.when(i < NUNIT // NBUF - 1)
                def _():
                    _fire(nxt, rows_bufs[jn], sems[jn])
            _drain(u, rows_bufs[j], sems[j])
            acc = _compute(u, rows_bufs[j], acc)
        return acc

    acc = lax.fori_loop(0, NUNIT // NBUF, _step, jnp.zeros((16,), jnp.float32))

    sum_v[...] = acc
    pltpu.sync_copy(out_v, out_hbm.at[pl.ds(b0, BPW)])
    pltpu.sync_copy(sum_v, sums_hbm.at[wid])


def _normalize_body(e_ref, s_ref, o_ref):
    total = jnp.sum(s_ref[...])
    scale = jnp.float32(B * KP1 / N_ROWS) / total
    o_ref[...] = e_ref[...] * scale


@jax.jit
def kernel(x, y, memory, idx):
    mesh = plsc.VectorSubcoreMesh(core_axis_name="c", subcore_axis_name="s")
    sc = pl.kernel(
        _sc_body,
        out_type=(
            jax.ShapeDtypeStruct((B, KP1), jnp.float32),
            jax.ShapeDtypeStruct((NW, 16), jnp.float32),
        ),
        mesh=mesh,
        compiler_params=pltpu.CompilerParams(
            use_tc_tiling_on_sc=False, needs_layout_passes=False),
        scratch_types=[
            pltpu.VMEM((BPW, D), jnp.float32),
            pltpu.VMEM((BPW,), jnp.int32),
            pltpu.VMEM((BPW, KP1), jnp.int32),
            pltpu.VMEM((LPU, D), jnp.float32),
            pltpu.VMEM((LPU, D), jnp.float32),
            pltpu.VMEM((LPU, D), jnp.float32),
            pltpu.VMEM((LPU, D), jnp.float32),
            pltpu.VMEM((BPW, KP1), jnp.float32),
            pltpu.VMEM((16,), jnp.float32),
            pltpu.SemaphoreType.DMA,
            pltpu.SemaphoreType.DMA,
            pltpu.SemaphoreType.DMA,
            pltpu.SemaphoreType.DMA,
        ],
    )
    from jax.experimental import layout as jlayout
    memory = jlayout.with_layout_constraint(
        memory, jlayout.Layout((0, 1), tiling=((8,), (1024,))))
    x_r = x.astype(jnp.bfloat16).astype(jnp.float32)
    e, sums = sc(x_r, y.astype(jnp.int32), memory, idx.astype(jnp.int32))
    out = pl.pallas_call(
        _normalize_body,
        out_shape=jax.ShapeDtypeStruct((B, KP1), jnp.float32),
    )(e, sums)
    return out
